# baseline (speedup 1.0000x reference)
import functools

import jax
import jax.numpy as jnp
from jax import lax
from jax.experimental import pallas as pl
from jax.experimental.pallas import tpu as pltpu
from jax.experimental.pallas import tpu_sc as plsc

N = 10000
E = 320000
D_IN = 128
D_H = 128
D_OUT = 64

NC = 2
NS = 16
NW = NC * NS
EPW = E // NW

CHUNK = 80
NCHUNK = EPW // CHUNK

N_PAD = 10240
RPT = N_PAD // NS
RPA = N // NS

DEG_GRP = 5

_F32 = jnp.float32


def _mesh():
    return plsc.VectorSubcoreMesh(core_axis_name="c", subcore_axis_name="s")


def _sc_degree(ei4):
    @functools.partial(
        pl.kernel,
        out_type=jax.ShapeDtypeStruct((NC, N_PAD), _F32),
        mesh=_mesh(),
        compiler_params=pltpu.CompilerParams(use_tc_tiling_on_sc=False),
        scratch_types=[
            pltpu.VMEM((NCHUNK, CHUNK), jnp.int32),
            pltpu.VMEM((CHUNK,), _F32),
            pltpu.VMEM((RPT,), _F32),
            pltpu.VMEM_SHARED((N_PAD,), _F32),
            pltpu.SemaphoreType.DMA,
        ],
    )
    def k(ei_hbm, out_hbm, didx2, ones, zbuf, acc, sem):
        c = lax.axis_index("c")
        s = lax.axis_index("s")
        wid = c * NS + s

        pltpu.sync_copy(ei_hbm.at[1, wid], didx2)

        @pl.loop(0, RPT // 16)
        def _(i):
            zbuf[pl.ds(i * 16, 16)] = jnp.zeros((16,), _F32)

        @pl.loop(0, CHUNK // 16)
        def _(i):
            ones[pl.ds(i * 16, 16)] = jnp.ones((16,), _F32)

        pltpu.sync_copy(zbuf, acc.at[pl.ds(s * RPT, RPT)])
        plsc.subcore_barrier()

        @pl.loop(0, NCHUNK // DEG_GRP)
        def _(j):
            i0 = j * DEG_GRP
            for b in range(DEG_GRP):
                pltpu.async_copy(ones, acc.at[didx2.at[i0 + b]], sem, add=True)
            for b in range(DEG_GRP):
                pltpu.make_async_copy(ones, acc.at[didx2.at[i0 + b]], sem).wait()

        plsc.subcore_barrier()
        pltpu.sync_copy(acc.at[pl.ds(s * RPT, RPT)],
                        out_hbm.at[c, pl.ds(s * RPT, RPT)])

    return k(ei4)


def _sc_aggregate(hp, zeros, ei4, d):
    ch = CHUNK
    nch = NCHUNK
    nbuf = 3 if d == D_H else 5
    nacc = N
    rpa = nacc // NS

    @functools.partial(
        pl.kernel,
        out_type=jax.ShapeDtypeStruct((NC, nacc, d), _F32),
        mesh=_mesh(),
        compiler_params=pltpu.CompilerParams(use_tc_tiling_on_sc=False),
        scratch_types=[
            pltpu.VMEM((nch, ch), jnp.int32),
            pltpu.VMEM((nch, ch), jnp.int32),
        ]
        + [pltpu.VMEM((ch, d), _F32) for _ in range(nbuf)]
        + [pltpu.VMEM_SHARED((nacc, d), _F32)]
        + [pltpu.SemaphoreType.DMA for _ in range(nbuf)],
    )
    def k(hp_hbm, z_hbm, ei_hbm, out_hbm, sidx2, didx2, *rest):
        rows = rest[:nbuf]
        acc = rest[nbuf]
        gsem = rest[nbuf + 1:]
        c = lax.axis_index("c")
        s = lax.axis_index("s")
        wid = c * NS + s

        pltpu.sync_copy(ei_hbm.at[0, wid], sidx2)
        pltpu.sync_copy(ei_hbm.at[1, wid], didx2)

        @pl.when(c == 0)
        def _():
            pltpu.sync_copy(hp_hbm.at[pl.ds(s * rpa, rpa)],
                            acc.at[pl.ds(s * rpa, rpa)])

        @pl.when(c == 1)
        def _():
            pltpu.sync_copy(z_hbm.at[pl.ds(s * rpa, rpa)],
                            acc.at[pl.ds(s * rpa, rpa)])

        plsc.subcore_barrier()

        def wait_gather(i, b):
            pltpu.make_async_copy(
                hp_hbm.at[sidx2.at[i]], rows[b], gsem[b]).wait()

        def scatter(i, b):
            pltpu.sync_copy(rows[b], acc.at[didx2.at[i]], add=True)

        def gather(i, b):
            pltpu.async_copy(hp_hbm.at[sidx2.at[i]], rows[b], gsem[b])

        for b in range(nbuf):
            gather(b, b)

        main_rounds = (nch - nbuf) // nbuf
        rem = (nch - nbuf) - main_rounds * nbuf

        @pl.loop(0, main_rounds)
        def _(j):
            i0 = j * nbuf
            for b in range(nbuf):
                wait_gather(i0 + b, b)
                scatter(i0 + b, b)
                gather(i0 + nbuf + b, b)

        base = main_rounds * nbuf
        for t in range(rem):
            i = base + t
            b = i % nbuf
            wait_gather(i, b)
            scatter(i, b)
            gather(i + nbuf, b)
        for t in range(nbuf):
            i = base + rem + t
            b = i % nbuf
            wait_gather(i, b)
            scatter(i, b)

        plsc.subcore_barrier()
        pltpu.sync_copy(acc.at[pl.ds(s * rpa, rpa)],
                        out_hbm.at[c, pl.ds(s * rpa, rpa)])

    return k(hp, zeros, ei4)


def _dinv_col(degp):
    return lax.rsqrt(degp[0, :N] + degp[1, :N] + 1.0)[:, None]


def _tc_first(x, w, degp):
    def body(x_ref, w_ref, degp_ref, out_ref):
        dinv = _dinv_col(degp_ref[...])
        h = jnp.dot(x_ref[...].astype(jnp.bfloat16),
                    w_ref[...].astype(jnp.bfloat16),
                    preferred_element_type=_F32)
        out_ref[...] = h * dinv

    return pl.pallas_call(
        body, out_shape=jax.ShapeDtypeStruct((N, D_H), _F32)
    )(x, w, degp)


def _tc_mid(degp, aggp, b, g, be, w):
    def body(degp_ref, aggp_ref, b_ref, g_ref, be_ref, w_ref, out_ref):
        dinv = _dinv_col(degp_ref[...])
        t = (aggp_ref[0, :N] + aggp_ref[1, :N]) * dinv
        t = t + b_ref[...]
        mu = jnp.mean(t, axis=0, keepdims=True)
        var = jnp.mean((t - mu) ** 2, axis=0, keepdims=True)
        t = (t - mu) * lax.rsqrt(var + 1e-5) * g_ref[...] + be_ref[...]
        t = jnp.maximum(t, 0.0)
        out_ref[...] = jnp.dot(t.astype(jnp.bfloat16),
                               w_ref[...].astype(jnp.bfloat16),
                               preferred_element_type=_F32) * dinv

    d_next = w.shape[1]
    return pl.pallas_call(
        body, out_shape=jax.ShapeDtypeStruct((N, d_next), _F32)
    )(degp, aggp, b, g, be, w)


def _tc_final(degp, aggp, b):
    def body(degp_ref, aggp_ref, b_ref, out_ref):
        dinv = _dinv_col(degp_ref[...])
        t = (aggp_ref[0, :N] + aggp_ref[1, :N]) * dinv
        t = t + b_ref[...]
        m = jnp.max(t, axis=1, keepdims=True)
        e = jnp.exp(t - m)
        out_ref[...] = e / jnp.sum(e, axis=1, keepdims=True)

    return pl.pallas_call(
        body, out_shape=jax.ShapeDtypeStruct((N, D_OUT), _F32)
    )(degp, aggp, b)


def kernel(x, edge_index, W1, b1, W2, b2, W3, b3, g1, be1, g2, be2):
    ei4 = edge_index.reshape(2, NW, NCHUNK, CHUNK)
    zeros_h = jnp.zeros((N, D_H), _F32)
    zeros_o = jnp.zeros((N, D_OUT), _F32)

    degp = _sc_degree(ei4)
    h1p = _tc_first(x, W1, degp)
    agg1 = _sc_aggregate(h1p, zeros_h, ei4, D_H)
    h2p = _tc_mid(degp, agg1, b1, g1, be1, W2)
    agg2 = _sc_aggregate(h2p, zeros_h, ei4, D_H)
    h3p = _tc_mid(degp, agg2, b2, g2, be2, W3)
    agg3 = _sc_aggregate(h3p, zeros_o, ei4, D_OUT)
    return _tc_final(degp, agg3, b3)


# speedup vs baseline: 1.0017x; 1.0017x over previous
import functools

import jax
import jax.numpy as jnp
from jax import lax
from jax.experimental import pallas as pl
from jax.experimental.pallas import tpu as pltpu
from jax.experimental.pallas import tpu_sc as plsc

N = 10000
E = 320000
D_IN = 128
D_H = 128
D_OUT = 64

NC = 2
NS = 16
NW = NC * NS
EPW = E // NW

CHUNK = 80
NCHUNK = EPW // CHUNK

N_PAD = 10240
RPT = N_PAD // NS
RPA = N // NS

DEG_GRP = 5

_F32 = jnp.float32


def _mesh():
    return plsc.VectorSubcoreMesh(core_axis_name="c", subcore_axis_name="s")


def _sc_degree(ei4):
    @functools.partial(
        pl.kernel,
        out_type=jax.ShapeDtypeStruct((NC, N_PAD), _F32),
        mesh=_mesh(),
        compiler_params=pltpu.CompilerParams(use_tc_tiling_on_sc=False),
        scratch_types=[
            pltpu.VMEM((NCHUNK, CHUNK), jnp.int32),
            pltpu.VMEM((CHUNK,), _F32),
            pltpu.VMEM((RPT,), _F32),
            pltpu.VMEM_SHARED((N_PAD,), _F32),
            pltpu.SemaphoreType.DMA,
        ],
    )
    def k(ei_hbm, out_hbm, didx2, ones, zbuf, acc, sem):
        c = lax.axis_index("c")
        s = lax.axis_index("s")
        wid = c * NS + s

        pltpu.sync_copy(ei_hbm.at[1, wid], didx2)

        @pl.loop(0, RPT // 16)
        def _(i):
            zbuf[pl.ds(i * 16, 16)] = jnp.zeros((16,), _F32)

        @pl.loop(0, CHUNK // 16)
        def _(i):
            ones[pl.ds(i * 16, 16)] = jnp.ones((16,), _F32)

        pltpu.sync_copy(zbuf, acc.at[pl.ds(s * RPT, RPT)])
        plsc.subcore_barrier()

        @pl.loop(0, NCHUNK // DEG_GRP)
        def _(j):
            i0 = j * DEG_GRP
            for b in range(DEG_GRP):
                pltpu.async_copy(ones, acc.at[didx2.at[i0 + b]], sem, add=True)
            for b in range(DEG_GRP):
                pltpu.make_async_copy(ones, acc.at[didx2.at[i0 + b]], sem).wait()

        plsc.subcore_barrier()
        pltpu.sync_copy(acc.at[pl.ds(s * RPT, RPT)],
                        out_hbm.at[c, pl.ds(s * RPT, RPT)])

    return k(ei4)


def _sc_aggregate(hp, zeros, ei4, d):
    ch = CHUNK
    nch = NCHUNK
    nbuf = 3 if d == D_H else 5
    nacc = N
    rpa = nacc // NS

    @functools.partial(
        pl.kernel,
        out_type=jax.ShapeDtypeStruct((NC, nacc, d), _F32),
        mesh=_mesh(),
        compiler_params=pltpu.CompilerParams(use_tc_tiling_on_sc=False),
        scratch_types=[
            pltpu.VMEM((nch, ch), jnp.int32),
            pltpu.VMEM((nch, ch), jnp.int32),
        ]
        + [pltpu.VMEM((ch, d), _F32) for _ in range(nbuf)]
        + [pltpu.VMEM_SHARED((nacc, d), _F32)]
        + [pltpu.SemaphoreType.DMA for _ in range(nbuf)],
    )
    def k(hp_hbm, z_hbm, ei_hbm, out_hbm, sidx2, didx2, *rest):
        rows = rest[:nbuf]
        acc = rest[nbuf]
        gsem = rest[nbuf + 1:]
        c = lax.axis_index("c")
        s = lax.axis_index("s")
        wid = c * NS + s

        pltpu.sync_copy(ei_hbm.at[0, wid], sidx2)
        pltpu.sync_copy(ei_hbm.at[1, wid], didx2)

        @pl.when(c == 0)
        def _():
            pltpu.sync_copy(hp_hbm.at[pl.ds(s * rpa, rpa)],
                            acc.at[pl.ds(s * rpa, rpa)])

        @pl.when(c == 1)
        def _():
            pltpu.sync_copy(z_hbm.at[pl.ds(s * rpa, rpa)],
                            acc.at[pl.ds(s * rpa, rpa)])

        plsc.subcore_barrier()

        def wait_gather(i, b):
            pltpu.make_async_copy(
                hp_hbm.at[sidx2.at[i]], rows[b], gsem[b]).wait()

        def scatter(i, b):
            pltpu.sync_copy(rows[b], acc.at[didx2.at[i]], add=True)

        def gather(i, b):
            pltpu.async_copy(hp_hbm.at[sidx2.at[i]], rows[b], gsem[b])

        for b in range(nbuf):
            gather(b, b)

        main_rounds = (nch - nbuf) // nbuf
        rem = (nch - nbuf) - main_rounds * nbuf

        @pl.loop(0, main_rounds)
        def _(j):
            i0 = j * nbuf
            for b in range(nbuf):
                wait_gather(i0 + b, b)
                scatter(i0 + b, b)
                gather(i0 + nbuf + b, b)

        base = main_rounds * nbuf
        for t in range(rem):
            i = base + t
            b = i % nbuf
            wait_gather(i, b)
            scatter(i, b)
            gather(i + nbuf, b)
        for t in range(nbuf):
            i = base + rem + t
            b = i % nbuf
            wait_gather(i, b)
            scatter(i, b)

        plsc.subcore_barrier()
        pltpu.sync_copy(acc.at[pl.ds(s * rpa, rpa)],
                        out_hbm.at[c, pl.ds(s * rpa, rpa)])

    return k(hp, zeros, ei4)


def _dinv_col(degp):
    return lax.rsqrt(degp[0, :N] + degp[1, :N] + 1.0)[:, None]


def _tc_first(x, w, degp):
    def body(x_ref, w_ref, degp_ref, out_ref):
        dinv = _dinv_col(degp_ref[...])
        h = jnp.dot(x_ref[...], w_ref[...], preferred_element_type=_F32)
        out_ref[...] = h * dinv

    return pl.pallas_call(
        body, out_shape=jax.ShapeDtypeStruct((N, D_H), _F32)
    )(x, w, degp)


def _tc_mid(degp, aggp, b, g, be, w):
    def body(degp_ref, aggp_ref, b_ref, g_ref, be_ref, w_ref, out_ref):
        dinv = _dinv_col(degp_ref[...])
        t = (aggp_ref[0, :N] + aggp_ref[1, :N]) * dinv
        t = t + b_ref[...]
        mu = jnp.mean(t, axis=0, keepdims=True)
        var = jnp.mean((t - mu) ** 2, axis=0, keepdims=True)
        t = (t - mu) * lax.rsqrt(var + 1e-5) * g_ref[...] + be_ref[...]
        t = jnp.maximum(t, 0.0)
        out_ref[...] = jnp.dot(t, w_ref[...], preferred_element_type=_F32) * dinv

    d_next = w.shape[1]
    return pl.pallas_call(
        body, out_shape=jax.ShapeDtypeStruct((N, d_next), _F32)
    )(degp, aggp, b, g, be, w)


def _tc_final(degp, aggp, b):
    def body(degp_ref, aggp_ref, b_ref, out_ref):
        dinv = _dinv_col(degp_ref[...])
        t = (aggp_ref[0, :N] + aggp_ref[1, :N]) * dinv
        t = t + b_ref[...]
        m = jnp.max(t, axis=1, keepdims=True)
        e = jnp.exp(t - m)
        out_ref[...] = e / jnp.sum(e, axis=1, keepdims=True)

    return pl.pallas_call(
        body, out_shape=jax.ShapeDtypeStruct((N, D_OUT), _F32)
    )(degp, aggp, b)


def kernel(x, edge_index, W1, b1, W2, b2, W3, b3, g1, be1, g2, be2):
    ei4 = edge_index.reshape(2, NW, NCHUNK, CHUNK)
    zeros_h = jnp.zeros((N, D_H), _F32)
    zeros_o = jnp.zeros((N, D_OUT), _F32)

    degp = _sc_degree(ei4)
    h1p = _tc_first(x, W1, degp)
    agg1 = _sc_aggregate(h1p, zeros_h, ei4, D_H)
    h2p = _tc_mid(degp, agg1, b1, g1, be1, W2)
    agg2 = _sc_aggregate(h2p, zeros_h, ei4, D_H)
    h3p = _tc_mid(degp, agg2, b2, g2, be2, W3)
    agg3 = _sc_aggregate(h3p, zeros_o, ei4, D_OUT)
    return _tc_final(degp, agg3, b3)
